# Initial kernel scaffold; baseline (speedup 1.0000x reference)
import jax
import jax.numpy as jnp
from jax.experimental import pallas as pl


def kernel(classifications, anchors, annotations):
    raise NotImplementedError("write your pallas kernel here")


# speedup vs baseline: 1.9364x; 1.9364x over previous
import jax
import jax.numpy as jnp
from jax.experimental import pallas as pl
from jax.experimental.pallas import tpu as pltpu

_NB = 2000


def _fl_kernel(cls_ref, anc_ref, annT_ref, out_ref, acc_ref):
    j = pl.program_id(0)
    nb = pl.program_id(1)
    nj = pl.num_programs(0)
    nnb = pl.num_programs(1)

    @pl.when(jnp.logical_and(j == 0, nb == 0))
    def _():
        acc_ref[0] = 0.0

    @pl.when(nb == 0)
    def _():
        acc_ref[1] = 0.0
        acc_ref[2] = 0.0

    x = jnp.clip(cls_ref[0], 1e-4, 1.0 - 1e-4)
    a = anc_ref[0, :, 0:1]
    ann = annT_ref[0]
    b0 = ann[0:1, :]
    b1 = ann[1:2, :]
    b2 = ann[2:3, :]
    length = b1 - b0

    comb = jnp.logical_and(a >= b0, a <= b1)
    inf = jnp.float32(jnp.inf)
    minlen = jnp.min(jnp.where(comb, length, inf), axis=1, keepdims=True)
    pos = minlen < inf
    gio = jax.lax.broadcasted_iota(jnp.int32, comb.shape, 1)
    tied = jnp.logical_and(comb, length == minlen)
    gsel = jnp.min(jnp.where(tied, gio, comb.shape[1]), axis=1, keepdims=True)
    clsv = jnp.sum(jnp.where(gio == gsel, b2, 0.0), axis=1, keepdims=True)
    clsi = clsv.astype(jnp.int32)

    kio = jax.lax.broadcasted_iota(jnp.int32, x.shape, 1)
    tmask = jnp.logical_and(pos, kio == clsi)

    base = 0.75 * (x * x) * (-jnp.log(1.0 - x))

    xsel = jnp.sum(jnp.where(tmask, x, 0.0), axis=1, keepdims=True)
    has = jnp.any(tmask, axis=1, keepdims=True)
    xs = jnp.where(has, xsel, 0.5)
    one_m = 1.0 - xs
    corr = jnp.where(
        has,
        0.25 * (one_m * one_m) * (-jnp.log(xs))
        - 0.75 * (xs * xs) * (-jnp.log(one_m)),
        0.0,
    )

    acc_ref[1] += jnp.sum(base) + jnp.sum(corr)
    acc_ref[2] += jnp.sum(pos.astype(jnp.float32))

    @pl.when(nb == nnb - 1)
    def _():
        acc_ref[0] += acc_ref[1] / jnp.maximum(acc_ref[2], 1.0)

    @pl.when(jnp.logical_and(j == nj - 1, nb == nnb - 1))
    def _():
        out_ref[...] = jnp.full((1, 1), acc_ref[0], dtype=jnp.float32)


def kernel(classifications, anchors, annotations):
    B, N, K = classifications.shape
    annT = jnp.transpose(annotations, (0, 2, 1))
    G = annT.shape[2]
    out = pl.pallas_call(
        _fl_kernel,
        grid=(B, N // _NB),
        in_specs=[
            pl.BlockSpec((1, _NB, K), lambda j, nb: (j, nb, 0)),
            pl.BlockSpec((1, _NB, 2), lambda j, nb: (0, nb, 0)),
            pl.BlockSpec((1, 3, G), lambda j, nb: (j, 0, 0)),
        ],
        out_specs=pl.BlockSpec((1, 1), lambda j, nb: (0, 0)),
        out_shape=jax.ShapeDtypeStruct((1, 1), jnp.float32),
        scratch_shapes=[pltpu.SMEM((4,), jnp.float32)],
    )(classifications, anchors, annT)
    return out[0, 0]


# speedup vs baseline: 1.9800x; 1.0225x over previous
import jax
import jax.numpy as jnp
from jax.experimental import pallas as pl
from jax.experimental.pallas import tpu as pltpu

_NB = 5000


def _fl_kernel(cls_ref, anc_ref, annT_ref, out_ref, acc_ref):
    j = pl.program_id(0)
    nb = pl.program_id(1)
    nj = pl.num_programs(0)
    nnb = pl.num_programs(1)

    @pl.when(jnp.logical_and(j == 0, nb == 0))
    def _():
        acc_ref[0] = 0.0

    @pl.when(nb == 0)
    def _():
        acc_ref[1] = 0.0
        acc_ref[2] = 0.0

    x = jnp.clip(cls_ref[0], 1e-4, 1.0 - 1e-4)
    a = anc_ref[0, :, 0:1]
    ann = annT_ref[0]
    b0 = ann[0:1, :]
    b1 = ann[1:2, :]
    b2 = ann[2:3, :]
    length = b1 - b0

    comb = jnp.logical_and(a >= b0, a <= b1)
    inf = jnp.float32(jnp.inf)
    minlen = jnp.min(jnp.where(comb, length, inf), axis=1, keepdims=True)
    pos = minlen < inf
    gio = jax.lax.broadcasted_iota(jnp.int32, comb.shape, 1)
    tied = jnp.logical_and(comb, length == minlen)
    gsel = jnp.min(jnp.where(tied, gio, comb.shape[1]), axis=1, keepdims=True)
    clsv = jnp.sum(jnp.where(gio == gsel, b2, 0.0), axis=1, keepdims=True)
    clsi = clsv.astype(jnp.int32)

    kio = jax.lax.broadcasted_iota(jnp.int32, x.shape, 1)
    tmask = jnp.logical_and(pos, kio == clsi)

    base = 0.75 * (x * x) * (-jnp.log(1.0 - x))

    xsel = jnp.sum(jnp.where(tmask, x, 0.0), axis=1, keepdims=True)
    has = jnp.any(tmask, axis=1, keepdims=True)
    xs = jnp.where(has, xsel, 0.5)
    one_m = 1.0 - xs
    corr = jnp.where(
        has,
        0.25 * (one_m * one_m) * (-jnp.log(xs))
        - 0.75 * (xs * xs) * (-jnp.log(one_m)),
        0.0,
    )

    acc_ref[1] += jnp.sum(base) + jnp.sum(corr)
    acc_ref[2] += jnp.sum(pos.astype(jnp.float32))

    @pl.when(nb == nnb - 1)
    def _():
        acc_ref[0] += acc_ref[1] / jnp.maximum(acc_ref[2], 1.0)

    @pl.when(jnp.logical_and(j == nj - 1, nb == nnb - 1))
    def _():
        out_ref[...] = jnp.full((1, 1), acc_ref[0], dtype=jnp.float32)


def kernel(classifications, anchors, annotations):
    B, N, K = classifications.shape
    annT = jnp.transpose(annotations, (0, 2, 1))
    G = annT.shape[2]
    out = pl.pallas_call(
        _fl_kernel,
        grid=(B, N // _NB),
        in_specs=[
            pl.BlockSpec((1, _NB, K), lambda j, nb: (j, nb, 0)),
            pl.BlockSpec((1, _NB, 2), lambda j, nb: (0, nb, 0)),
            pl.BlockSpec((1, 3, G), lambda j, nb: (j, 0, 0)),
        ],
        out_specs=pl.BlockSpec((1, 1), lambda j, nb: (0, 0)),
        out_shape=jax.ShapeDtypeStruct((1, 1), jnp.float32),
        scratch_shapes=[pltpu.SMEM((4,), jnp.float32)],
    )(classifications, anchors, annT)
    return out[0, 0]


# speedup vs baseline: 3.4996x; 1.7675x over previous
import jax
import jax.numpy as jnp
from jax.experimental import pallas as pl
from jax.experimental.pallas import tpu as pltpu

_NB = 2000


def _fl_kernel(cls_ref, ancr_ref, ann_ref, out_ref, acc_ref):
    j = pl.program_id(0)
    nb = pl.program_id(1)
    nj = pl.num_programs(0)
    nnb = pl.num_programs(1)

    @pl.when(jnp.logical_and(j == 0, nb == 0))
    def _():
        acc_ref[0] = 0.0

    @pl.when(nb == 0)
    def _():
        acc_ref[1] = 0.0
        acc_ref[2] = 0.0

    x = jnp.clip(cls_ref[0].T, 1e-4, 1.0 - 1e-4)
    a = ancr_ref[0]
    ann = ann_ref[0]
    b0 = ann[:, 0:1]
    b1 = ann[:, 1:2]
    b2 = ann[:, 2:3]
    length = b1 - b0

    comb = jnp.logical_and(a >= b0, a <= b1)
    inf = jnp.float32(jnp.inf)
    minlen = jnp.min(jnp.where(comb, length, inf), axis=0, keepdims=True)
    pos = minlen < inf
    gio = jax.lax.broadcasted_iota(jnp.int32, comb.shape, 0)
    tied = jnp.logical_and(comb, length == minlen)
    gsel = jnp.min(jnp.where(tied, gio, comb.shape[0]), axis=0, keepdims=True)
    clsv = jnp.sum(jnp.where(gio == gsel, b2, 0.0), axis=0, keepdims=True)
    clsi = clsv.astype(jnp.int32)

    kio = jax.lax.broadcasted_iota(jnp.int32, x.shape, 0)
    tmask = jnp.logical_and(pos, kio == clsi)

    base = 0.75 * (x * x) * (-jnp.log(1.0 - x))

    xsel = jnp.sum(jnp.where(tmask, x, 0.0), axis=0, keepdims=True)
    has = jnp.any(tmask, axis=0, keepdims=True)
    xs = jnp.where(has, xsel, 0.5)
    one_m = 1.0 - xs
    corr = jnp.where(
        has,
        0.25 * (one_m * one_m) * (-jnp.log(xs))
        - 0.75 * (xs * xs) * (-jnp.log(one_m)),
        0.0,
    )

    acc_ref[1] += jnp.sum(base) + jnp.sum(corr)
    acc_ref[2] += jnp.sum(pos.astype(jnp.float32))

    @pl.when(nb == nnb - 1)
    def _():
        acc_ref[0] += acc_ref[1] / jnp.maximum(acc_ref[2], 1.0)

    @pl.when(jnp.logical_and(j == nj - 1, nb == nnb - 1))
    def _():
        out_ref[...] = jnp.full((1, 1), acc_ref[0], dtype=jnp.float32)


def kernel(classifications, anchors, annotations):
    B, N, K = classifications.shape
    anchor_row = anchors[0, :, 0].reshape(N // _NB, 1, _NB)
    G = annotations.shape[1]
    out = pl.pallas_call(
        _fl_kernel,
        grid=(B, N // _NB),
        in_specs=[
            pl.BlockSpec((1, _NB, K), lambda j, nb: (j, nb, 0)),
            pl.BlockSpec((1, 1, _NB), lambda j, nb: (nb, 0, 0)),
            pl.BlockSpec((1, G, 3), lambda j, nb: (j, 0, 0)),
        ],
        out_specs=pl.BlockSpec((1, 1), lambda j, nb: (0, 0)),
        out_shape=jax.ShapeDtypeStruct((1, 1), jnp.float32),
        scratch_shapes=[pltpu.SMEM((4,), jnp.float32)],
    )(classifications, anchor_row, annotations)
    return out[0, 0]


# speedup vs baseline: 4.5105x; 1.2888x over previous
import jax
import jax.numpy as jnp
from jax.experimental import pallas as pl
from jax.experimental.pallas import tpu as pltpu

_NB = 4000


def _fl_kernel(cls_ref, ancr_ref, ann_ref, out_ref, acc_ref, lacc_ref, pacc_ref):
    j = pl.program_id(0)
    nb = pl.program_id(1)
    nj = pl.num_programs(0)
    nnb = pl.num_programs(1)

    @pl.when(jnp.logical_and(j == 0, nb == 0))
    def _():
        acc_ref[0] = 0.0

    @pl.when(nb == 0)
    def _():
        lacc_ref[...] = jnp.zeros_like(lacc_ref)
        pacc_ref[...] = jnp.zeros_like(pacc_ref)

    x = jnp.clip(cls_ref[0].T, 1e-4, 1.0 - 1e-4)
    K = x.shape[0]
    a = ancr_ref[0]
    ann = ann_ref[0]
    b0 = ann[:, 0:1]
    b1 = ann[:, 1:2]
    b2 = ann[:, 2:3]
    length = b1 - b0

    comb = jnp.logical_and(a >= b0, a <= b1)
    inf = jnp.float32(jnp.inf)
    minlen = jnp.min(jnp.where(comb, length, inf), axis=0, keepdims=True)
    pos = minlen < inf
    gio = jax.lax.broadcasted_iota(jnp.int32, comb.shape, 0)
    tied = jnp.logical_and(comb, length == minlen)
    gsel = jnp.min(jnp.where(tied, gio, comb.shape[0]), axis=0, keepdims=True)
    clsv = jnp.sum(jnp.where(gio == gsel, b2, 0.0), axis=0, keepdims=True)
    clsi = jnp.where(pos, clsv.astype(jnp.int32), -1)

    kio = jax.lax.broadcasted_iota(jnp.int32, x.shape, 0)
    tmask = kio == clsi

    base_row = jnp.sum(0.75 * (x * x) * (-jnp.log(1.0 - x)),
                       axis=0, keepdims=True)

    xsel = jnp.sum(jnp.where(tmask, x, 0.0), axis=0, keepdims=True)
    has = jnp.logical_and(clsi >= 0, clsi < K)
    xs = jnp.where(has, xsel, 0.5)
    one_m = 1.0 - xs
    corr = jnp.where(
        has,
        0.25 * (one_m * one_m) * (-jnp.log(xs))
        - 0.75 * (xs * xs) * (-jnp.log(one_m)),
        0.0,
    )

    lacc_ref[...] += base_row + corr
    pacc_ref[...] += pos.astype(jnp.float32)

    @pl.when(nb == nnb - 1)
    def _():
        bsum = jnp.sum(lacc_ref[...])
        npos = jnp.sum(pacc_ref[...])
        acc_ref[0] += bsum / jnp.maximum(npos, 1.0)

    @pl.when(jnp.logical_and(j == nj - 1, nb == nnb - 1))
    def _():
        out_ref[...] = jnp.full((1, 1), acc_ref[0], dtype=jnp.float32)


def kernel(classifications, anchors, annotations):
    B, N, K = classifications.shape
    anchor_row = anchors[0, :, 0].reshape(N // _NB, 1, _NB)
    G = annotations.shape[1]
    out = pl.pallas_call(
        _fl_kernel,
        grid=(B, N // _NB),
        in_specs=[
            pl.BlockSpec((1, _NB, K), lambda j, nb: (j, nb, 0)),
            pl.BlockSpec((1, 1, _NB), lambda j, nb: (nb, 0, 0)),
            pl.BlockSpec((1, G, 3), lambda j, nb: (j, 0, 0)),
        ],
        out_specs=pl.BlockSpec((1, 1), lambda j, nb: (0, 0)),
        out_shape=jax.ShapeDtypeStruct((1, 1), jnp.float32),
        scratch_shapes=[
            pltpu.SMEM((4,), jnp.float32),
            pltpu.VMEM((1, _NB), jnp.float32),
            pltpu.VMEM((1, _NB), jnp.float32),
        ],
    )(classifications, anchor_row, annotations)
    return out[0, 0]


# speedup vs baseline: 4.5613x; 1.0113x over previous
import jax
import jax.numpy as jnp
from jax.experimental import pallas as pl
from jax.experimental.pallas import tpu as pltpu

_NB = 4000


_LN2 = 0.6931471805599453


def _fl_kernel(cls_ref, ancr_ref, ann_ref, out_ref, acc_ref, lacc_ref, cacc_ref, pacc_ref):
    j = pl.program_id(0)
    nb = pl.program_id(1)
    nj = pl.num_programs(0)
    nnb = pl.num_programs(1)

    @pl.when(jnp.logical_and(j == 0, nb == 0))
    def _():
        acc_ref[0] = 0.0

    @pl.when(nb == 0)
    def _():
        lacc_ref[...] = jnp.zeros_like(lacc_ref)
        cacc_ref[...] = jnp.zeros_like(cacc_ref)
        pacc_ref[...] = jnp.zeros_like(pacc_ref)

    x = jnp.clip(cls_ref[0].T, 1e-4, 1.0 - 1e-4)
    K = x.shape[0]
    a = ancr_ref[0]
    ann = ann_ref[0]
    b0 = ann[:, 0:1]
    b1 = ann[:, 1:2]
    b2 = ann[:, 2:3]
    length = b1 - b0

    comb = jnp.logical_and(a >= b0, a <= b1)
    inf = jnp.float32(jnp.inf)
    minlen = jnp.min(jnp.where(comb, length, inf), axis=0, keepdims=True)
    pos = minlen < inf
    gio = jax.lax.broadcasted_iota(jnp.int32, comb.shape, 0)
    tied = jnp.logical_and(comb, length == minlen)
    gsel = jnp.min(jnp.where(tied, gio, comb.shape[0]), axis=0, keepdims=True)
    clsv = jnp.sum(jnp.where(gio == gsel, b2, 0.0), axis=0, keepdims=True)
    clsi = jnp.where(pos, clsv.astype(jnp.int32), -1)

    kio = jax.lax.broadcasted_iota(jnp.int32, x.shape, 0)
    tmask = kio == clsi

    y = (x * x) * jnp.log2(1.0 - x)
    y8 = jnp.sum(y.reshape(K // 8, 8, -1), axis=0)

    xsel = jnp.sum(jnp.where(tmask, x, 0.0), axis=0, keepdims=True)
    has = jnp.logical_and(clsi >= 0, clsi < K)
    xs = jnp.where(has, xsel, 0.5)
    one_m = 1.0 - xs
    corr = jnp.where(
        has,
        0.25 * (one_m * one_m) * (-jnp.log(xs))
        - 0.75 * (xs * xs) * (-jnp.log(one_m)),
        0.0,
    )

    lacc_ref[...] += y8
    cacc_ref[...] += corr
    pacc_ref[...] += pos.astype(jnp.float32)

    @pl.when(nb == nnb - 1)
    def _():
        bsum = (-0.75 * _LN2) * jnp.sum(lacc_ref[...]) + jnp.sum(cacc_ref[...])
        npos = jnp.sum(pacc_ref[...])
        acc_ref[0] += bsum / jnp.maximum(npos, 1.0)

    @pl.when(jnp.logical_and(j == nj - 1, nb == nnb - 1))
    def _():
        out_ref[...] = jnp.full((1, 1), acc_ref[0], dtype=jnp.float32)


def kernel(classifications, anchors, annotations):
    B, N, K = classifications.shape
    anchor_row = anchors[0, :, 0].reshape(N // _NB, 1, _NB)
    G = annotations.shape[1]
    out = pl.pallas_call(
        _fl_kernel,
        grid=(B, N // _NB),
        in_specs=[
            pl.BlockSpec((1, _NB, K), lambda j, nb: (j, nb, 0)),
            pl.BlockSpec((1, 1, _NB), lambda j, nb: (nb, 0, 0)),
            pl.BlockSpec((1, G, 3), lambda j, nb: (j, 0, 0)),
        ],
        out_specs=pl.BlockSpec((1, 1), lambda j, nb: (0, 0)),
        out_shape=jax.ShapeDtypeStruct((1, 1), jnp.float32),
        scratch_shapes=[
            pltpu.SMEM((4,), jnp.float32),
            pltpu.VMEM((8, _NB), jnp.float32),
            pltpu.VMEM((1, _NB), jnp.float32),
            pltpu.VMEM((1, _NB), jnp.float32),
        ],
    )(classifications, anchor_row, annotations)
    return out[0, 0]
